# baseline (speedup 1.0000x reference)
import jax
import jax.numpy as jnp
from jax.experimental import pallas as pl
from jax.experimental.pallas import tpu as pltpu

_BM = 512
_NBLK = 16


def _mm_kernel(x_ref, w_ref, b_ref, o_ref, xbf_ref, w_bf_ref):
    i = pl.program_id(0)

    @pl.when(i == 0)
    def _():
        w_bf_ref[...] = w_ref[...].astype(jnp.bfloat16)

    s = i % 2
    xbf_ref[s] = x_ref[...].astype(jnp.bfloat16)
    acc = jnp.dot(
        xbf_ref[1 - s],
        w_bf_ref[...],
        preferred_element_type=jnp.float32,
    )
    o_ref[...] = acc + b_ref[...]


def kernel(x, W_router, b_router, W0, b0):
    B, S, D = x.shape
    M = B * S
    x2 = x.reshape(M, D)
    b2 = b0.reshape(1, D)

    out = pl.pallas_call(
        _mm_kernel,
        grid=(_NBLK + 1,),
        in_specs=[
            pl.BlockSpec((_BM, D), lambda i: (jnp.minimum(i, _NBLK - 1), 0)),
            pl.BlockSpec((D, D), lambda i: (0, 0)),
            pl.BlockSpec((1, D), lambda i: (0, 0)),
        ],
        out_specs=pl.BlockSpec((_BM, D), lambda i: (jnp.maximum(i - 1, 0), 0)),
        out_shape=jax.ShapeDtypeStruct((M, D), jnp.float32),
        scratch_shapes=[
            pltpu.VMEM((2, _BM, D), jnp.bfloat16),
            pltpu.VMEM((D, D), jnp.bfloat16),
        ],
        compiler_params=pltpu.CompilerParams(
            dimension_semantics=("arbitrary",),
        ),
    )(x2, W0, b2)
    return out.reshape(B, S, D)


# speedup vs baseline: 1.0029x; 1.0029x over previous
import jax
import jax.numpy as jnp
from jax.experimental import pallas as pl
from jax.experimental.pallas import tpu as pltpu

_BM = 512


def _cast_kernel(w_ref, o_ref):
    o_ref[...] = w_ref[...].astype(jnp.bfloat16)


def _mm_kernel(x_ref, w_ref, b_ref, o_ref):
    acc = jnp.dot(
        x_ref[...].astype(jnp.bfloat16),
        w_ref[...],
        preferred_element_type=jnp.float32,
    )
    o_ref[...] = acc + b_ref[...]


def kernel(x, W_router, b_router, W0, b0):
    B, S, D = x.shape
    M = B * S
    x2 = x.reshape(M, D)
    b2 = b0.reshape(1, D)

    w_bf = pl.pallas_call(
        _cast_kernel,
        out_shape=jax.ShapeDtypeStruct((D, D), jnp.bfloat16),
    )(W0)

    out = pl.pallas_call(
        _mm_kernel,
        grid=(M // _BM,),
        in_specs=[
            pl.BlockSpec((_BM, D), lambda i: (i, 0)),
            pl.BlockSpec((D, D), lambda i: (0, 0)),
            pl.BlockSpec((1, D), lambda i: (0, 0)),
        ],
        out_specs=pl.BlockSpec((_BM, D), lambda i: (i, 0)),
        out_shape=jax.ShapeDtypeStruct((M, D), jnp.float32),
        compiler_params=pltpu.CompilerParams(
            dimension_semantics=("arbitrary",),
        ),
    )(x2, w_bf, b2)
    return out.reshape(B, S, D)


# speedup vs baseline: 1.0031x; 1.0002x over previous
import jax
import jax.numpy as jnp
from jax.experimental import pallas as pl
from jax.experimental.pallas import tpu as pltpu

_BM = 512


def _cast_kernel(w_ref, o_ref):
    o_ref[...] = w_ref[...].astype(jnp.bfloat16)


def _mm_kernel(x_ref, w_ref, b_ref, o_ref):
    acc = jnp.dot(
        x_ref[...].astype(jnp.bfloat16),
        w_ref[...],
        preferred_element_type=jnp.float32,
    )
    o_ref[...] = acc + b_ref[...]


def kernel(x, W_router, b_router, W0, b0):
    B, S, D = x.shape
    M = B * S
    x2 = x.reshape(M, D)
    b2 = b0.reshape(1, D)

    w_bf = pl.pallas_call(
        _cast_kernel,
        out_shape=jax.ShapeDtypeStruct((D, D), jnp.bfloat16),
    )(W0)

    out = pl.pallas_call(
        _mm_kernel,
        grid=(M // _BM,),
        in_specs=[
            pl.BlockSpec((_BM, D), lambda i: (i, 0)),
            pl.BlockSpec((D, D), lambda i: (0, 0)),
            pl.BlockSpec((1, D), lambda i: (0, 0)),
        ],
        out_specs=pl.BlockSpec((_BM, D), lambda i: (i, 0)),
        out_shape=jax.ShapeDtypeStruct((M, D), jnp.float32),
        compiler_params=pltpu.CompilerParams(
            dimension_semantics=("parallel",),
        ),
    )(x2, w_bf, b2)
    return out.reshape(B, S, D)


# speedup vs baseline: 1.0739x; 1.0706x over previous
import jax
import jax.numpy as jnp
from jax.experimental import pallas as pl
from jax.experimental.pallas import tpu as pltpu

_BM = 1024


def _mm_kernel(x_ref, w_ref, b_ref, o_ref, w_bf_ref):
    i = pl.program_id(0)

    @pl.when(i == 0)
    def _():
        w_bf_ref[...] = w_ref[...].astype(jnp.bfloat16)

    acc = jnp.dot(
        x_ref[...].astype(jnp.bfloat16),
        w_bf_ref[...],
        preferred_element_type=jnp.float32,
    )
    o_ref[...] = acc + b_ref[...]


def kernel(x, W_router, b_router, W0, b0):
    B, S, D = x.shape
    M = B * S
    x2 = x.reshape(M, D)
    b2 = b0.reshape(1, D)

    out = pl.pallas_call(
        _mm_kernel,
        grid=(M // _BM,),
        in_specs=[
            pl.BlockSpec((_BM, D), lambda i: (i, 0)),
            pl.BlockSpec((D, D), lambda i: (0, 0)),
            pl.BlockSpec((1, D), lambda i: (0, 0)),
        ],
        out_specs=pl.BlockSpec((_BM, D), lambda i: (i, 0)),
        out_shape=jax.ShapeDtypeStruct((M, D), jnp.float32),
        scratch_shapes=[pltpu.VMEM((D, D), jnp.bfloat16)],
        compiler_params=pltpu.CompilerParams(
            dimension_semantics=("arbitrary",),
            vmem_limit_bytes=100 * 1024 * 1024,
        ),
    )(x2, W0, b2)
    return out.reshape(B, S, D)
